# Initial kernel scaffold; baseline (speedup 1.0000x reference)
import functools

import jax
import jax.numpy as jnp
from jax import lax
from jax.experimental import pallas as pl
from jax.experimental.pallas import tpu as pltpu
from jax.experimental.pallas import tpu_sc as plsc

NUM_CORES = 2
NUM_SUBCORES = 16
LANES = 16
EPS = 1e-10
CHUNK = 400


def _proj_body(nf_ref, rhs_ref, out_ref):
    out_ref[0] = jnp.dot(nf_ref[...], rhs_ref[0],
                         preferred_element_type=jnp.float32)


def _final_body(za_ref, zb_ref, wo_ref, out_ref):
    d = wo_ref.shape[1]
    out_ref[...] = (
        jnp.dot(za_ref[0], wo_ref[0:d, :], preferred_element_type=jnp.float32)
        + jnp.dot(zb_ref[0], wo_ref[d:2 * d, :],
                  preferred_element_type=jnp.float32))


def _mm_stack(nf, rhs_stack, bn):
    n, d = nf.shape
    g = rhs_stack.shape[0]
    return pl.pallas_call(
        _proj_body,
        grid=(g, n // bn),
        in_specs=[
            pl.BlockSpec((bn, d), lambda gi, nb: (nb, 0)),
            pl.BlockSpec((1, d, d), lambda gi, nb: (gi, 0, 0)),
        ],
        out_specs=pl.BlockSpec((1, bn, d), lambda gi, nb: (gi, nb, 0)),
        out_shape=jax.ShapeDtypeStruct((g, n, d), jnp.float32),
    )(nf, rhs_stack)


def _make_sc_kernel(n_nodes, n_edges, d, n_rel):
    ept = n_edges // NUM_SUBCORES
    nch = ept // CHUNK
    zrows = n_nodes // NUM_SUBCORES
    scale = 1.0 / float(jnp.sqrt(jnp.float32(d * NUM_CORES)))
    assert ept * NUM_SUBCORES == n_edges and nch * CHUNK == ept
    assert zrows * NUM_SUBCORES == n_nodes and CHUNK % LANES == 0

    def body(src_hbm, dst_hbm, et_hbm, kh_hbm, vh_hbm, qh_hbm, zout_hbm,
             src_c, dst_c, et_c, kidx_c, qidx_c, wbuf, rows_a, rows_b,
             numer_t, denom_l, denom_sh, z_sh, sem_a, sem_b):
        c = lax.axis_index("c")
        s = lax.axis_index("s")
        tile_base = s * ept
        zero16 = jnp.zeros((LANES,), jnp.float32)

        def z0_loop(i, _):
            for j in range(d // LANES):
                rows_a[i, pl.ds(j * LANES, LANES)] = zero16
            return 0
        lax.fori_loop(0, CHUNK, z0_loop, 0)

        @pl.when(s == 0)
        def _():
            def dz_loop(i, _):
                denom_l[pl.ds(i * LANES, LANES)] = zero16
                return 0
            lax.fori_loop(0, n_nodes // LANES, dz_loop, 0)
            pltpu.sync_copy(denom_l, denom_sh)

        zc = 125
        for k in range(zrows // zc):
            pltpu.sync_copy(rows_a.at[pl.ds(0, zc)],
                            z_sh.at[pl.ds(s * zrows + k * zc, zc)])
        plsc.subcore_barrier()

        def phase1(i, _):
            base = tile_base + i * CHUNK
            pltpu.sync_copy(src_hbm.at[pl.ds(base, CHUNK)], src_c)
            pltpu.sync_copy(dst_hbm.at[pl.ds(base, CHUNK)], dst_c)
            pltpu.sync_copy(et_hbm.at[pl.ds(base, CHUNK)], et_c)

            def idx_loop(j, _):
                sl = pl.ds(j * LANES, LANES)
                kidx_c[sl] = (et_c[sl] + c * n_rel) * n_nodes + src_c[sl]
                qidx_c[sl] = dst_c[sl] + c * n_nodes
                return 0
            lax.fori_loop(0, CHUNK // LANES, idx_loop, 0)

            cp_k = pltpu.async_copy(kh_hbm.at[kidx_c], rows_a, sem_a)
            cp_q = pltpu.async_copy(qh_hbm.at[qidx_c], rows_b, sem_b)
            cp_k.wait()
            cp_q.wait()

            def edge_loop(e, _):
                acc = rows_a[e, pl.ds(0, LANES)] * rows_b[e, pl.ds(0, LANES)]
                for j in range(1, d // LANES):
                    sl = pl.ds(j * LANES, LANES)
                    acc = acc + rows_a[e, sl] * rows_b[e, sl]
                sc = jnp.sum(acc) * scale
                sc = jnp.maximum(sc, 0.0)
                numer_t[i * CHUNK + e] = sc * sc + EPS
                return 0
            lax.fori_loop(0, CHUNK, edge_loop, 0)

            pltpu.sync_copy(numer_t.at[pl.ds(i * CHUNK, CHUNK)],
                            denom_sh.at[dst_c], add=True)
            return 0
        lax.fori_loop(0, nch, phase1, 0)
        plsc.subcore_barrier()

        pltpu.sync_copy(denom_sh, denom_l)

        def phase2(i, _):
            base = tile_base + i * CHUNK
            pltpu.sync_copy(src_hbm.at[pl.ds(base, CHUNK)], src_c)
            pltpu.sync_copy(dst_hbm.at[pl.ds(base, CHUNK)], dst_c)
            pltpu.sync_copy(et_hbm.at[pl.ds(base, CHUNK)], et_c)

            def widx_loop(j, _):
                sl = pl.ds(j * LANES, LANES)
                kidx_c[sl] = (et_c[sl] + c * n_rel) * n_nodes + src_c[sl]
                dv = plsc.load_gather(denom_l, [dst_c[sl]])
                nv = numer_t[pl.ds(i * CHUNK + j * LANES, LANES)]
                wbuf[sl] = nv / dv
                return 0
            lax.fori_loop(0, CHUNK // LANES, widx_loop, 0)

            pltpu.async_copy(vh_hbm.at[kidx_c], rows_a, sem_a).wait()

            def scale_loop(e, _):
                w = wbuf[e]
                for j in range(d // LANES):
                    sl = pl.ds(j * LANES, LANES)
                    rows_b[e, sl] = rows_a[e, sl] * w
                return 0
            lax.fori_loop(0, CHUNK, scale_loop, 0)

            pltpu.sync_copy(rows_b, z_sh.at[dst_c], add=True)
            return 0
        lax.fori_loop(0, nch, phase2, 0)
        plsc.subcore_barrier()

        pltpu.sync_copy(z_sh.at[pl.ds(s * zrows, zrows)],
                        zout_hbm.at[pl.ds(c * n_nodes + s * zrows, zrows)])

    mesh = plsc.VectorSubcoreMesh(core_axis_name="c", subcore_axis_name="s",
                                  num_cores=NUM_CORES,
                                  num_subcores=NUM_SUBCORES)
    return pl.kernel(
        body,
        out_type=jax.ShapeDtypeStruct((NUM_CORES * n_nodes, d), jnp.float32),
        mesh=mesh,
        scratch_types=[
            pltpu.VMEM((CHUNK,), jnp.int32),
            pltpu.VMEM((CHUNK,), jnp.int32),
            pltpu.VMEM((CHUNK,), jnp.int32),
            pltpu.VMEM((CHUNK,), jnp.int32),
            pltpu.VMEM((CHUNK,), jnp.int32),
            pltpu.VMEM((CHUNK,), jnp.float32),
            pltpu.VMEM((CHUNK, d), jnp.float32),
            pltpu.VMEM((CHUNK, d), jnp.float32),
            pltpu.VMEM((ept,), jnp.float32),
            pltpu.VMEM((n_nodes,), jnp.float32),
            pltpu.VMEM_SHARED((n_nodes,), jnp.float32),
            pltpu.VMEM_SHARED((n_nodes, d), jnp.float32),
            pltpu.SemaphoreType.DMA,
            pltpu.SemaphoreType.DMA,
        ],
    )


@jax.jit
def kernel(node_feature, edge_index, edge_type, WQ, WK, WV, WO):
    n, d = node_feature.shape
    n_rel = WK.shape[0]
    hd = WQ.shape[1]
    h = hd // d
    e = edge_index.shape[1]
    assert h == NUM_CORES

    src = edge_index[0]
    dst = edge_index[1]

    wk_stack = WK.reshape(n_rel, d, h, d).transpose(2, 0, 1, 3)
    wk_stack = wk_stack.reshape(h * n_rel, d, d)
    wv_stack = WV.reshape(n_rel, d, h, d).transpose(2, 0, 1, 3)
    wv_stack = wv_stack.reshape(h * n_rel, d, d)
    wq_stack = WQ.reshape(d, h, d).transpose(1, 0, 2)

    bn = 1000
    kh = _mm_stack(node_feature, wk_stack, bn).reshape(h * n_rel * n, d)
    vh = _mm_stack(node_feature, wv_stack, bn).reshape(h * n_rel * n, d)
    qh = _mm_stack(node_feature, wq_stack, bn).reshape(h * n, d)

    sc_fn = _make_sc_kernel(n, e, d, n_rel)
    zout = sc_fn(src, dst, edge_type, kh, vh, qh)
    zr = zout.reshape(h, n, d)

    out = pl.pallas_call(
        _final_body,
        grid=(n // bn,),
        in_specs=[
            pl.BlockSpec((1, bn, d), lambda nb: (0, nb, 0)),
            pl.BlockSpec((1, bn, d), lambda nb: (1, nb, 0)),
            pl.BlockSpec((h * d, d), lambda nb: (0, 0)),
        ],
        out_specs=pl.BlockSpec((bn, d), lambda nb: (nb, 0)),
        out_shape=jax.ShapeDtypeStruct((n, d), jnp.float32),
    )(zr, zr, WO)
    return out


# speedup vs baseline: 15.1063x; 15.1063x over previous
import functools

import jax
import jax.numpy as jnp
from jax import lax
from jax.experimental import pallas as pl
from jax.experimental.pallas import tpu as pltpu
from jax.experimental.pallas import tpu_sc as plsc

NUM_CORES = 2
NUM_SUBCORES = 16
LANES = 16
EPS = 1e-10
CHUNK = 80


def _proj_body(nf_ref, rhs_ref, out_ref):
    out_ref[0] = jnp.dot(nf_ref[...], rhs_ref[0],
                         preferred_element_type=jnp.float32)


def _final_body(za_ref, zb_ref, wo_ref, out_ref):
    d = wo_ref.shape[1]
    out_ref[...] = (
        jnp.dot(za_ref[0], wo_ref[0:d, :], preferred_element_type=jnp.float32)
        + jnp.dot(zb_ref[0], wo_ref[d:2 * d, :],
                  preferred_element_type=jnp.float32))


def _kidx_body(n_nodes, src_ref, et_ref, out_ref):
    out_ref[...] = et_ref[...] * n_nodes + src_ref[...]


def _mm_stack(nf, rhs_stack, bn):
    n, d = nf.shape
    g = rhs_stack.shape[0]
    return pl.pallas_call(
        _proj_body,
        grid=(g, n // bn),
        in_specs=[
            pl.BlockSpec((bn, d), lambda gi, nb: (nb, 0)),
            pl.BlockSpec((1, d, d), lambda gi, nb: (gi, 0, 0)),
        ],
        out_specs=pl.BlockSpec((1, bn, d), lambda gi, nb: (gi, nb, 0)),
        out_shape=jax.ShapeDtypeStruct((g, n, d), jnp.float32),
    )(nf, rhs_stack)


def _make_sc_kernel(n_nodes, n_edges, d, n_rel):
    ept = n_edges // NUM_SUBCORES
    nch = ept // CHUNK
    scale = 1.0 / (float(d * NUM_CORES) ** 0.5)
    assert ept * NUM_SUBCORES == n_edges and nch * CHUNK == ept
    assert CHUNK % LANES == 0 and n_nodes % 1000 == 0

    def body(kidx_hbm, dst_hbm, kh_hbm, vh_hbm, qh_hbm, zout_hbm,
             kidx_c, dst_c, qidx_c, wbuf, rows_a, rows_b,
             numer_t, denom_l, denom_sh, z_sh, sem_a, sem_b):
        c = lax.axis_index("c")
        s = lax.axis_index("s")
        tile_base = s * ept
        koff = c * (n_rel * n_nodes)
        zero16 = jnp.zeros((LANES,), jnp.float32)
        iota16 = lax.iota(jnp.int32, LANES)

        def z0_loop(i, _):
            for j in range(d // LANES):
                rows_a[i, pl.ds(j * LANES, LANES)] = zero16
            return 0
        lax.fori_loop(0, CHUNK, z0_loop, 0)

        @pl.when(s == 0)
        def _():
            def dz_loop(i, _):
                denom_l[pl.ds(i * LANES, LANES)] = zero16
                return 0
            lax.fori_loop(0, n_nodes // LANES, dz_loop, 0)
            pltpu.sync_copy(denom_l, denom_sh)

        @pl.when(s < n_nodes // 1000)
        def _():
            done = 0
            while done < 1000:
                zc = min(CHUNK, 1000 - done)
                pltpu.sync_copy(rows_a.at[pl.ds(0, zc)],
                                z_sh.at[pl.ds(s * 1000 + done, zc)])
                done += zc
        plsc.subcore_barrier()

        def phase1(i, _):
            base = tile_base + i * CHUNK
            pltpu.sync_copy(kidx_hbm.at[pl.ds(base, CHUNK)], kidx_c)
            pltpu.sync_copy(dst_hbm.at[pl.ds(base, CHUNK)], dst_c)

            def idx_loop(j, _):
                sl = pl.ds(j * LANES, LANES)
                kidx_c[sl] = kidx_c[sl] + koff
                qidx_c[sl] = dst_c[sl] + c * n_nodes
                return 0
            lax.fori_loop(0, CHUNK // LANES, idx_loop, 0)

            cp_k = pltpu.async_copy(kh_hbm.at[kidx_c], rows_a, sem_a)
            cp_q = pltpu.async_copy(qh_hbm.at[qidx_c], rows_b, sem_b)
            cp_k.wait()
            cp_q.wait()

            def group_loop(g, _):
                def e_loop(e2, sv):
                    e = g * LANES + e2
                    acc = (rows_a[e, pl.ds(0, LANES)]
                           * rows_b[e, pl.ds(0, LANES)])
                    for j in range(1, d // LANES):
                        sl = pl.ds(j * LANES, LANES)
                        acc = acc + rows_a[e, sl] * rows_b[e, sl]
                    sc = jnp.maximum(jnp.sum(acc) * scale, 0.0)
                    return jnp.where(iota16 == e2, sc * sc + EPS, sv)
                sv = lax.fori_loop(0, LANES, e_loop,
                                   jnp.zeros((LANES,), jnp.float32))
                numer_t[pl.ds(i * CHUNK + g * LANES, LANES)] = sv
                return 0
            lax.fori_loop(0, CHUNK // LANES, group_loop, 0)

            pltpu.sync_copy(numer_t.at[pl.ds(i * CHUNK, CHUNK)],
                            denom_sh.at[dst_c], add=True)
            return 0
        lax.fori_loop(0, nch, phase1, 0)
        plsc.subcore_barrier()

        pltpu.sync_copy(denom_sh, denom_l)

        def phase2(i, _):
            base = tile_base + i * CHUNK
            pltpu.sync_copy(kidx_hbm.at[pl.ds(base, CHUNK)], kidx_c)
            pltpu.sync_copy(dst_hbm.at[pl.ds(base, CHUNK)], dst_c)

            def widx_loop(j, _):
                sl = pl.ds(j * LANES, LANES)
                kidx_c[sl] = kidx_c[sl] + koff
                dv = plsc.load_gather(denom_l, [dst_c[sl]])
                nv = numer_t[pl.ds(i * CHUNK + j * LANES, LANES)]
                wbuf[sl] = nv / dv
                return 0
            lax.fori_loop(0, CHUNK // LANES, widx_loop, 0)

            pltpu.async_copy(vh_hbm.at[kidx_c], rows_a, sem_a).wait()

            def scale_loop(e, _):
                wsplat = plsc.load_gather(
                    wbuf, [jnp.full((LANES,), e, jnp.int32)])
                for j in range(d // LANES):
                    sl = pl.ds(j * LANES, LANES)
                    rows_b[e, sl] = rows_a[e, sl] * wsplat
                return 0
            lax.fori_loop(0, CHUNK, scale_loop, 0)

            pltpu.sync_copy(rows_b, z_sh.at[dst_c], add=True)
            return 0
        lax.fori_loop(0, nch, phase2, 0)
        plsc.subcore_barrier()

        @pl.when(s < n_nodes // 1000)
        def _():
            pltpu.sync_copy(z_sh.at[pl.ds(s * 1000, 1000)],
                            zout_hbm.at[pl.ds(c * n_nodes + s * 1000, 1000)])

    mesh = plsc.VectorSubcoreMesh(core_axis_name="c", subcore_axis_name="s",
                                  num_cores=NUM_CORES,
                                  num_subcores=NUM_SUBCORES)
    return pl.kernel(
        body,
        out_type=jax.ShapeDtypeStruct((NUM_CORES * n_nodes, d), jnp.float32),
        mesh=mesh,
        compiler_params=pltpu.CompilerParams(needs_layout_passes=False),
        scratch_types=[
            pltpu.VMEM((CHUNK,), jnp.int32),
            pltpu.VMEM((CHUNK,), jnp.int32),
            pltpu.VMEM((CHUNK,), jnp.int32),
            pltpu.VMEM((CHUNK,), jnp.float32),
            pltpu.VMEM((CHUNK, d), jnp.float32),
            pltpu.VMEM((CHUNK, d), jnp.float32),
            pltpu.VMEM((ept,), jnp.float32),
            pltpu.VMEM((n_nodes,), jnp.float32),
            pltpu.VMEM_SHARED((n_nodes,), jnp.float32),
            pltpu.VMEM_SHARED((n_nodes, d), jnp.float32),
            pltpu.SemaphoreType.DMA,
            pltpu.SemaphoreType.DMA,
        ],
    )


@jax.jit
def kernel(node_feature, edge_index, edge_type, WQ, WK, WV, WO):
    n, d = node_feature.shape
    n_rel = WK.shape[0]
    hd = WQ.shape[1]
    h = hd // d
    e = edge_index.shape[1]
    assert h == NUM_CORES

    src2 = edge_index[0].reshape(e // 128, 128)
    et2 = edge_type.reshape(e // 128, 128)
    dst = edge_index[1]

    kidx = pl.pallas_call(
        functools.partial(_kidx_body, n),
        out_shape=jax.ShapeDtypeStruct((e // 128, 128), jnp.int32),
    )(src2, et2).reshape(e)

    wk_stack = WK.reshape(n_rel, d, h, d).transpose(2, 0, 1, 3)
    wk_stack = wk_stack.reshape(h * n_rel, d, d)
    wv_stack = WV.reshape(n_rel, d, h, d).transpose(2, 0, 1, 3)
    wv_stack = wv_stack.reshape(h * n_rel, d, d)
    wq_stack = WQ.reshape(d, h, d).transpose(1, 0, 2)

    bn = 1000
    kh = _mm_stack(node_feature, wk_stack, bn).reshape(h * n_rel * n, d)
    vh = _mm_stack(node_feature, wv_stack, bn).reshape(h * n_rel * n, d)
    qh = _mm_stack(node_feature, wq_stack, bn).reshape(h * n, d)

    sc_fn = _make_sc_kernel(n, e, d, n_rel)
    zout = sc_fn(kidx, dst, kh, vh, qh)
    zr = zout.reshape(h, n, d)

    out = pl.pallas_call(
        _final_body,
        grid=(n // bn,),
        in_specs=[
            pl.BlockSpec((1, bn, d), lambda nb: (0, nb, 0)),
            pl.BlockSpec((1, bn, d), lambda nb: (1, nb, 0)),
            pl.BlockSpec((h * d, d), lambda nb: (0, 0)),
        ],
        out_specs=pl.BlockSpec((bn, d), lambda nb: (nb, 0)),
        out_shape=jax.ShapeDtypeStruct((n, d), jnp.float32),
    )(zr, zr, WO)
    return out
